# baseline (device time: 180214 ns/iter reference)
import numpy as np

import jax
import jax.numpy as jnp
from jax import lax
from jax.experimental import pallas as pl
from jax.experimental.pallas import tpu as pltpu

N_DEV = 4
B, SQ, SKV_SH, DH = 2, 512, 512, 64
H_SH = 8
HID = H_SH * DH
SKV = N_DEV * SKV_SH
D_OUT = 768
BLK = 64
PAIR = 2 * DH

BF = jnp.bfloat16
F32 = jnp.float32


def _global_mask() -> np.ndarray:
    qb = (np.arange(SQ) // BLK)[:, None]
    kb = (np.arange(SKV) // BLK)[None, :]
    m = (qb == kb) | (kb == 0) | ((qb + kb) % 3 == 0)
    return m.astype(np.float32)


def kernel(x, Wq, K_ext, V_ext, Wo):
    K2 = K_ext.reshape(B, SKV_SH, N_DEV * HID)
    V2 = V_ext.reshape(B, SKV_SH, N_DEV * HID)
    mask = jnp.asarray(_global_mask(), dtype=BF)

    def body(x_ref, wq_ref, k_ref, v_ref, wo_ref, mask_ref, out_ref,
             kvsend, kvall, qbuf, mbuf, lbuf, cbuf, osend, orecv,
             kv_send_sems, kv_recv_sems, o_send_sems, o_recv_sems):
        my = lax.axis_index("i")

        kv_rdmas = {}
        for d in range(1, N_DEV):
            dst = lax.rem(my + d, N_DEV)
            kvsend[d - 1, 0] = k_ref[:, :, pl.ds(dst * HID, HID)].astype(BF)
            kvsend[d - 1, 1] = v_ref[:, :, pl.ds(dst * HID, HID)].astype(BF)
            r = pltpu.make_async_remote_copy(
                src_ref=kvsend.at[d - 1],
                dst_ref=kvall.at[d - 1],
                send_sem=kv_send_sems.at[d - 1],
                recv_sem=kv_recv_sems.at[d - 1],
                device_id=(dst,),
                device_id_type=pl.DeviceIdType.MESH,
            )
            r.start()
            kv_rdmas[d] = r

        wo_bf = wo_ref[:, :].astype(BF)
        wq_bf = wq_ref[:, :].astype(BF)
        for b in range(B):
            qbuf[b] = lax.dot_general(
                x_ref[b].astype(BF), wq_bf, (((1,), (0,)), ((), ())),
                preferred_element_type=F32).astype(BF)

        def run_phase(d, is_first, is_last):
            src = lax.rem(my - d + N_DEV, N_DEV)
            for b in range(B):
                def pair_step(hp, carry, b=b, d=d,
                              is_first=is_first, is_last=is_last, src=src):
                    hs = hp * PAIR
                    q2 = qbuf[b, :, pl.ds(hs, PAIR)]
                    if d == 0:
                        kc2 = k_ref[b, :, pl.ds(my * HID + hs, PAIR)].astype(BF)
                        vc2 = v_ref[b, :, pl.ds(my * HID + hs, PAIR)].astype(BF)
                    else:
                        kc2 = kvall[d - 1, 0, b, :, pl.ds(hs, PAIR)]
                        vc2 = kvall[d - 1, 1, b, :, pl.ds(hs, PAIR)]
                    mk = mask_ref[:, pl.ds(src * SKV_SH, SKV_SH)]
                    if not is_first:
                        ml2 = mbuf[b, :, pl.ds(hs, PAIR)]
                        ll2 = lbuf[b, :, pl.ds(hs, PAIR)]
                        cc2 = cbuf[b, :, pl.ds(hs, PAIR)]
                    cs, ms, ls = [], [], []
                    for sub in range(2):
                        lo, hi = sub * DH, (sub + 1) * DH
                        q = q2[:, lo:hi]
                        kc = kc2[:, lo:hi]
                        vc = vc2[:, lo:hi]
                        s = lax.dot_general(q, kc,
                                            (((1,), (1,)), ((), ())),
                                            preferred_element_type=F32) * 0.125
                        s = jnp.where(mk > 0.5, s, -1e9)
                        smax = s.max(axis=1, keepdims=True)
                        if is_first:
                            m_new = smax
                            p = jnp.exp(s - m_new)
                            l = p.sum(axis=1, keepdims=True)
                            ctx = lax.dot_general(
                                p.astype(BF), vc, (((1,), (0,)), ((), ())),
                                preferred_element_type=F32)
                        else:
                            m_old = ml2[:, lo:lo + 1]
                            m_new = jnp.maximum(m_old, smax)
                            alpha = jnp.exp(m_old - m_new)
                            p = jnp.exp(s - m_new)
                            l = ll2[:, lo:lo + 1] * alpha + p.sum(
                                axis=1, keepdims=True)
                            ctx = cc2[:, lo:hi] * alpha + lax.dot_general(
                                p.astype(BF), vc, (((1,), (0,)), ((), ())),
                                preferred_element_type=F32)
                        if is_last:
                            ctx = ctx / l
                        cs.append(ctx)
                        ms.append(jnp.broadcast_to(m_new, (SQ, DH)))
                        ls.append(jnp.broadcast_to(l, (SQ, DH)))
                    cbuf[b, :, pl.ds(hs, PAIR)] = jnp.concatenate(cs, axis=1)
                    if not is_last:
                        mbuf[b, :, pl.ds(hs, PAIR)] = jnp.concatenate(
                            ms, axis=1)
                        lbuf[b, :, pl.ds(hs, PAIR)] = jnp.concatenate(
                            ls, axis=1)
                    return carry

                lax.fori_loop(0, H_SH // 2, pair_step, 0)

        run_phase(0, True, False)
        for d, is_last in ((1, False), (3, False), (2, True)):
            kv_rdmas[d].wait_recv()
            run_phase(d, False, is_last)

        o_rdmas = []
        for b in range(B):
            out_ref[b] = lax.dot_general(
                cbuf[b].astype(BF), wo_bf, (((1,), (0,)), ((), ())),
                preferred_element_type=F32)
            osend[b] = out_ref[b].astype(BF)
            for d in range(1, N_DEV):
                dst = lax.rem(my + d, N_DEV)
                r = pltpu.make_async_remote_copy(
                    src_ref=osend.at[b],
                    dst_ref=orecv.at[d - 1, b],
                    send_sem=o_send_sems.at[d - 1, b],
                    recv_sem=o_recv_sems.at[d - 1, b],
                    device_id=(dst,),
                    device_id_type=pl.DeviceIdType.MESH,
                )
                r.start()
                o_rdmas.append(r)

        for r in kv_rdmas.values():
            r.wait_send()
        for r in o_rdmas:
            r.wait_send()
            r.wait_recv()
        out_ref[:, :, :] = (out_ref[:, :, :]
                            + orecv[0].astype(F32)
                            + orecv[1].astype(F32)
                            + orecv[2].astype(F32))

    return pl.pallas_call(
        body,
        out_shape=jax.ShapeDtypeStruct((B, SQ, D_OUT), F32),
        in_specs=[pl.BlockSpec(memory_space=pltpu.VMEM)] * 6,
        out_specs=pl.BlockSpec(memory_space=pltpu.VMEM),
        scratch_shapes=[
            pltpu.VMEM((N_DEV - 1, 2, B, SKV_SH, HID), BF),
            pltpu.VMEM((N_DEV - 1, 2, B, SKV_SH, HID), BF),
            pltpu.VMEM((B, SQ, HID), BF),
            pltpu.VMEM((B, SQ, HID), F32),
            pltpu.VMEM((B, SQ, HID), F32),
            pltpu.VMEM((B, SQ, HID), F32),
            pltpu.VMEM((B, SQ, D_OUT), BF),
            pltpu.VMEM((N_DEV - 1, B, SQ, D_OUT), BF),
            pltpu.SemaphoreType.DMA((N_DEV - 1,)),
            pltpu.SemaphoreType.DMA((N_DEV - 1,)),
            pltpu.SemaphoreType.DMA((N_DEV - 1, B)),
            pltpu.SemaphoreType.DMA((N_DEV - 1, B)),
        ],
        compiler_params=pltpu.CompilerParams(
            vmem_limit_bytes=100 * 1024 * 1024,
        ),
    )(x, Wq, K2, V2, Wo, mask)


# device time: 143310 ns/iter; 1.2575x vs baseline; 1.2575x over previous
import numpy as np

import jax
import jax.numpy as jnp
from jax import lax
from jax.experimental import pallas as pl
from jax.experimental.pallas import tpu as pltpu

N_DEV = 4
B, SQ, SKV_SH, DH = 2, 512, 512, 64
H_SH = 8
HID = H_SH * DH
SKV = N_DEV * SKV_SH
D_OUT = 768
BLK = 64
PAIR = 2 * DH

BF = jnp.bfloat16
F32 = jnp.float32


def _global_mask() -> np.ndarray:
    qb = (np.arange(SQ) // BLK)[:, None]
    kb = (np.arange(SKV) // BLK)[None, :]
    m = (qb == kb) | (kb == 0) | ((qb + kb) % 3 == 0)
    return m.astype(np.float32)


def kernel(x, Wq, K_ext, V_ext, Wo):
    K2 = K_ext.reshape(B, SKV_SH, N_DEV * HID)
    V2 = V_ext.reshape(B, SKV_SH, N_DEV * HID)
    mask = jnp.asarray(_global_mask(), dtype=F32)

    def body(x_ref, wq_ref, k_ref, v_ref, wo_ref, mask_ref, out_ref,
             kvsend, kvall, qbuf, cbuf, osend, orecv,
             kv_send_sems, kv_recv_sems, o_send_sems, o_recv_sems):
        my = lax.axis_index("i")

        kv_rdmas = {}
        for d in range(1, N_DEV):
            dst = lax.rem(my + d, N_DEV)
            kvsend[d - 1, 0] = k_ref[:, :, pl.ds(dst * HID, HID)].astype(BF)
            kvsend[d - 1, 1] = v_ref[:, :, pl.ds(dst * HID, HID)].astype(BF)
            r = pltpu.make_async_remote_copy(
                src_ref=kvsend.at[d - 1],
                dst_ref=kvall.at[d - 1],
                send_sem=kv_send_sems.at[d - 1],
                recv_sem=kv_recv_sems.at[d - 1],
                device_id=(dst,),
                device_id_type=pl.DeviceIdType.MESH,
            )
            r.start()
            kv_rdmas[d] = r

        wo_bf = wo_ref[:, :].astype(BF)
        wq_bf = wq_ref[:, :].astype(BF)
        for b in range(B):
            qbuf[b] = lax.dot_general(
                x_ref[b].astype(BF), wq_bf, (((1,), (0,)), ((), ())),
                preferred_element_type=F32).astype(BF)

        for r in kv_rdmas.values():
            r.wait_recv()

        for b in range(B):
            def pair_step(hp, carry, b=b):
                hs = hp * PAIR
                q2 = qbuf[b, :, pl.ds(hs, PAIR)]
                chunks = []
                for d in range(N_DEV):
                    src = lax.rem(my - d + N_DEV, N_DEV)
                    if d == 0:
                        kc2 = k_ref[b, :,
                                    pl.ds(my * HID + hs, PAIR)].astype(BF)
                        vc2 = v_ref[b, :,
                                    pl.ds(my * HID + hs, PAIR)].astype(BF)
                    else:
                        kc2 = kvall[d - 1, 0, b, :, pl.ds(hs, PAIR)]
                        vc2 = kvall[d - 1, 1, b, :, pl.ds(hs, PAIR)]
                    mk = mask_ref[:, pl.ds(src * SKV_SH, SKV_SH)]
                    chunks.append((kc2, vc2, mk))
                outs = []
                for sub in range(2):
                    lo, hi = sub * DH, (sub + 1) * DH
                    q = q2[:, lo:hi]
                    l = jnp.zeros((SQ, 1), F32)
                    ctx = jnp.zeros((SQ, DH), F32)
                    for kc2, vc2, mk in chunks:
                        s = lax.dot_general(q, kc2[:, lo:hi],
                                            (((1,), (1,)), ((), ())),
                                            preferred_element_type=F32)
                        p = jnp.exp(s * 0.125) * mk
                        l = l + p.sum(axis=1, keepdims=True)
                        ctx = ctx + lax.dot_general(
                            p.astype(BF), vc2[:, lo:hi],
                            (((1,), (0,)), ((), ())),
                            preferred_element_type=F32)
                    outs.append(ctx / l)
                cbuf[b, :, pl.ds(hs, PAIR)] = jnp.concatenate(outs, axis=1)
                return carry

            lax.fori_loop(0, H_SH // 2, pair_step, 0)

        o_rdmas = []
        for b in range(B):
            out_ref[b] = lax.dot_general(
                cbuf[b].astype(BF), wo_bf, (((1,), (0,)), ((), ())),
                preferred_element_type=F32)
            osend[b] = out_ref[b].astype(BF)
            for d in range(1, N_DEV):
                dst = lax.rem(my + d, N_DEV)
                r = pltpu.make_async_remote_copy(
                    src_ref=osend.at[b],
                    dst_ref=orecv.at[d - 1, b],
                    send_sem=o_send_sems.at[d - 1, b],
                    recv_sem=o_recv_sems.at[d - 1, b],
                    device_id=(dst,),
                    device_id_type=pl.DeviceIdType.MESH,
                )
                r.start()
                o_rdmas.append(r)

        for r in kv_rdmas.values():
            r.wait_send()
        for r in o_rdmas:
            r.wait_send()
            r.wait_recv()
        out_ref[:, :, :] = (out_ref[:, :, :]
                            + orecv[0].astype(F32)
                            + orecv[1].astype(F32)
                            + orecv[2].astype(F32))

    return pl.pallas_call(
        body,
        out_shape=jax.ShapeDtypeStruct((B, SQ, D_OUT), F32),
        in_specs=[pl.BlockSpec(memory_space=pltpu.VMEM)] * 6,
        out_specs=pl.BlockSpec(memory_space=pltpu.VMEM),
        scratch_shapes=[
            pltpu.VMEM((N_DEV - 1, 2, B, SKV_SH, HID), BF),
            pltpu.VMEM((N_DEV - 1, 2, B, SKV_SH, HID), BF),
            pltpu.VMEM((B, SQ, HID), BF),
            pltpu.VMEM((B, SQ, HID), F32),
            pltpu.VMEM((B, SQ, D_OUT), BF),
            pltpu.VMEM((N_DEV - 1, B, SQ, D_OUT), BF),
            pltpu.SemaphoreType.DMA((N_DEV - 1,)),
            pltpu.SemaphoreType.DMA((N_DEV - 1,)),
            pltpu.SemaphoreType.DMA((N_DEV - 1, B)),
            pltpu.SemaphoreType.DMA((N_DEV - 1, B)),
        ],
        compiler_params=pltpu.CompilerParams(
            vmem_limit_bytes=100 * 1024 * 1024,
        ),
    )(x, Wq, K2, V2, Wo, mask)


# device time: 136894 ns/iter; 1.3164x vs baseline; 1.0469x over previous
import numpy as np

import jax
import jax.numpy as jnp
from jax import lax
from jax.experimental import pallas as pl
from jax.experimental.pallas import tpu as pltpu

N_DEV = 4
B, SQ, SKV_SH, DH = 2, 512, 512, 64
H_SH = 8
HID = H_SH * DH
SKV = N_DEV * SKV_SH
D_OUT = 768
BLK = 64
PAIR = 2 * DH
NQB = SQ // BLK
NKB = SKV // BLK

QCLS_BLOCKS = [[0, 3, 6], [1, 4, 7], [2, 5]]
QCLS_BASE = [0, 192, 384]
QLEN = [192, 192, 128]
KRES_BASE = [0, 704, 1408]
KLEN = [704, 704, 640]
PAIR_RES = [0, 2, 1]
PERMPOS = [QCLS_BASE[qb % 3] // BLK + qb // 3 for qb in range(NQB)]

BF = jnp.bfloat16
F32 = jnp.float32


def kernel(x, Wq, K_ext, V_ext, Wo):
    K2 = K_ext.reshape(B, SKV_SH, N_DEV * HID)
    V2 = V_ext.reshape(B, SKV_SH, N_DEV * HID)

    def body(x_ref, wq_ref, k_ref, v_ref, wo_ref, out_ref,
             kvsend, kvall, kg, vg, qbuf, cbuf, osend, orecv,
             kv_send_sems, kv_recv_sems, o_send_sems, o_recv_sems):
        my = lax.axis_index("i")

        kv_rdmas = {}
        for d in range(1, N_DEV):
            dst = lax.rem(my + d, N_DEV)
            kvsend[d - 1, 0] = k_ref[:, :, pl.ds(dst * HID, HID)].astype(BF)
            kvsend[d - 1, 1] = v_ref[:, :, pl.ds(dst * HID, HID)].astype(BF)
            r = pltpu.make_async_remote_copy(
                src_ref=kvsend.at[d - 1],
                dst_ref=kvall.at[d - 1],
                send_sem=kv_send_sems.at[d - 1],
                recv_sem=kv_recv_sems.at[d - 1],
                device_id=(dst,),
                device_id_type=pl.DeviceIdType.MESH,
            )
            r.start()
            kv_rdmas[d] = r

        wo_bf = wo_ref[:, :].astype(BF)
        wq_bf = wq_ref[:, :].astype(BF)
        for b in range(B):
            qp = lax.dot_general(
                x_ref[b].astype(BF), wq_bf, (((1,), (0,)), ((), ())),
                preferred_element_type=F32).astype(BF)
            pos = 0
            for r in range(3):
                for qb in QCLS_BLOCKS[r]:
                    qbuf[b, pos * BLK:(pos + 1) * BLK, :] = (
                        qp[qb * BLK:(qb + 1) * BLK, :])
                    pos += 1

        def assemble(d):
            src = lax.rem(my - d + N_DEV, N_DEV)
            for j in range(NQB):
                g = src * NQB + j
                off = (lax.rem(g, 3) * 11 + g // 3) * BLK
                rows = slice(j * BLK, (j + 1) * BLK)
                if d == 0:
                    kg[:, pl.ds(off, BLK), :] = (
                        k_ref[:, rows, pl.ds(my * HID, HID)].astype(BF))
                    vg[:, pl.ds(off, BLK), :] = (
                        v_ref[:, rows, pl.ds(my * HID, HID)].astype(BF))
                else:
                    kg[:, pl.ds(off, BLK), :] = kvall[d - 1, 0, :, rows, :]
                    vg[:, pl.ds(off, BLK), :] = kvall[d - 1, 1, :, rows, :]

        assemble(0)
        for d in (1, 3, 2):
            kv_rdmas[d].wait_recv()
            assemble(d)

        for b in range(B):
            def pair_step(hp, carry, b=b):
                hs = hp * PAIR
                q2 = qbuf[b, :, pl.ds(hs, PAIR)]
                k0 = kg[b, 0:BLK, pl.ds(hs, PAIR)]
                v0 = vg[b, 0:BLK, pl.ds(hs, PAIR)]
                outs = []
                lparts = [[], []]
                cparts = [[], []]
                for r in range(3):
                    r2 = PAIR_RES[r]
                    kr = kg[b, pl.ds(KRES_BASE[r2], KLEN[r2]),
                            pl.ds(hs, PAIR)]
                    vr = vg[b, pl.ds(KRES_BASE[r2], KLEN[r2]),
                            pl.ds(hs, PAIR)]
                    extras = []
                    if r != 0:
                        for qb in QCLS_BLOCKS[r]:
                            drow = (11 * (qb % 3) + qb // 3) * BLK
                            kd = kg[b, pl.ds(drow, BLK), pl.ds(hs, PAIR)]
                            vd = vg[b, pl.ds(drow, BLK), pl.ds(hs, PAIR)]
                            extras.append((kd, vd))
                    for sub in range(2):
                        lo, hi = sub * DH, (sub + 1) * DH
                        qr = q2[QCLS_BASE[r]:QCLS_BASE[r] + QLEN[r], lo:hi]
                        s = lax.dot_general(qr, kr[:, lo:hi],
                                            (((1,), (1,)), ((), ())),
                                            preferred_element_type=F32)
                        p = jnp.exp(s * 0.125)
                        lr = p.sum(axis=1, keepdims=True)
                        cr = lax.dot_general(p.astype(BF), vr[:, lo:hi],
                                             (((1,), (0,)), ((), ())),
                                             preferred_element_type=F32)
                        if r != 0:
                            les, ces = [], []
                            for idx, qb in enumerate(QCLS_BLOCKS[r]):
                                kd, vd = extras[idx]
                                qe = q2[QCLS_BASE[r] + idx * BLK:
                                        QCLS_BASE[r] + (idx + 1) * BLK,
                                        lo:hi]
                                ke = jnp.concatenate(
                                    [k0[:, lo:hi], kd[:, lo:hi]], axis=0)
                                ve = jnp.concatenate(
                                    [v0[:, lo:hi], vd[:, lo:hi]], axis=0)
                                se = lax.dot_general(
                                    qe, ke, (((1,), (1,)), ((), ())),
                                    preferred_element_type=F32)
                                pe = jnp.exp(se * 0.125)
                                les.append(pe.sum(axis=1, keepdims=True))
                                ces.append(lax.dot_general(
                                    pe.astype(BF), ve,
                                    (((1,), (0,)), ((), ())),
                                    preferred_element_type=F32))
                            lr = lr + jnp.concatenate(les, axis=0)
                            cr = cr + jnp.concatenate(ces, axis=0)
                        lparts[sub].append(lr)
                        cparts[sub].append(cr)
                for sub in range(2):
                    ctx_all = jnp.concatenate(cparts[sub], axis=0)
                    l_all = jnp.concatenate(lparts[sub], axis=0)
                    outs.append(ctx_all / l_all)
                cbuf[b, :, pl.ds(hs, PAIR)] = jnp.concatenate(outs, axis=1)
                return carry

            lax.fori_loop(0, H_SH // 2, pair_step, 0)

        o_rdmas = []
        for b in range(B):
            outp = lax.dot_general(
                cbuf[b].astype(BF), wo_bf, (((1,), (0,)), ((), ())),
                preferred_element_type=F32)
            for qb in range(NQB):
                pp = PERMPOS[qb]
                rows_val = outp[pp * BLK:(pp + 1) * BLK, :]
                out_ref[b, qb * BLK:(qb + 1) * BLK, :] = rows_val
                osend[b, qb * BLK:(qb + 1) * BLK, :] = rows_val.astype(BF)
            for d in range(1, N_DEV):
                dst = lax.rem(my + d, N_DEV)
                r = pltpu.make_async_remote_copy(
                    src_ref=osend.at[b],
                    dst_ref=orecv.at[d - 1, b],
                    send_sem=o_send_sems.at[d - 1, b],
                    recv_sem=o_recv_sems.at[d - 1, b],
                    device_id=(dst,),
                    device_id_type=pl.DeviceIdType.MESH,
                )
                r.start()
                o_rdmas.append(r)

        for r in kv_rdmas.values():
            r.wait_send()
        for r in o_rdmas:
            r.wait_send()
            r.wait_recv()
        out_ref[:, :, :] = (out_ref[:, :, :]
                            + orecv[0].astype(F32)
                            + orecv[1].astype(F32)
                            + orecv[2].astype(F32))

    return pl.pallas_call(
        body,
        out_shape=jax.ShapeDtypeStruct((B, SQ, D_OUT), F32),
        in_specs=[pl.BlockSpec(memory_space=pltpu.VMEM)] * 5,
        out_specs=pl.BlockSpec(memory_space=pltpu.VMEM),
        scratch_shapes=[
            pltpu.VMEM((N_DEV - 1, 2, B, SKV_SH, HID), BF),
            pltpu.VMEM((N_DEV - 1, 2, B, SKV_SH, HID), BF),
            pltpu.VMEM((B, SKV, HID), BF),
            pltpu.VMEM((B, SKV, HID), BF),
            pltpu.VMEM((B, SQ, HID), BF),
            pltpu.VMEM((B, SQ, HID), F32),
            pltpu.VMEM((B, SQ, D_OUT), BF),
            pltpu.VMEM((N_DEV - 1, B, SQ, D_OUT), BF),
            pltpu.SemaphoreType.DMA((N_DEV - 1,)),
            pltpu.SemaphoreType.DMA((N_DEV - 1,)),
            pltpu.SemaphoreType.DMA((N_DEV - 1, B)),
            pltpu.SemaphoreType.DMA((N_DEV - 1, B)),
        ],
        compiler_params=pltpu.CompilerParams(
            vmem_limit_bytes=100 * 1024 * 1024,
        ),
    )(x, Wq, K2, V2, Wo)


# device time: 116087 ns/iter; 1.5524x vs baseline; 1.1792x over previous
import numpy as np

import jax
import jax.numpy as jnp
from jax import lax
from jax.experimental import pallas as pl
from jax.experimental.pallas import tpu as pltpu

N_DEV = 4
B, SQ, SKV_SH, DH = 2, 512, 512, 64
H_SH = 8
HID = H_SH * DH
SKV = N_DEV * SKV_SH
D_OUT = 768
BLK = 64
PAIR = 2 * DH
NQB = SQ // BLK

QCLS_BLOCKS = [[0, 3, 6], [1, 4, 7], [2, 5]]
QCLS_BASE = [0, 192, 384]
QLEN = [192, 192, 128]
KRES_BASE = [0, 704, 1408]
KLEN = [704, 704, 640]
PAIR_RES = [0, 2, 1]
PERMPOS = [QCLS_BASE[qb % 3] // BLK + qb // 3 for qb in range(NQB)]

BF = jnp.bfloat16
F32 = jnp.float32


def kernel(x, Wq, K_ext, V_ext, Wo):
    K2 = K_ext.reshape(B, SKV_SH, N_DEV * HID)
    V2 = V_ext.reshape(B, SKV_SH, N_DEV * HID)

    def body(x_ref, wq_ref, k_ref, v_ref, wo_ref, out_ref,
             kvsend, kvall, kg, vg, qbuf, cbuf, osend, orecv,
             kv_send_sems, kv_recv_sems, o_send_sems, o_recv_sems):
        my = lax.axis_index("i")

        kv_rdmas = {}
        for b in range(B):
            for d in range(1, N_DEV):
                dst = lax.rem(my + d, N_DEV)
                kvsend[d - 1, b, 0] = (
                    k_ref[b, :, pl.ds(dst * HID, HID)].astype(BF))
                kvsend[d - 1, b, 1] = (
                    v_ref[b, :, pl.ds(dst * HID, HID)].astype(BF))
                r = pltpu.make_async_remote_copy(
                    src_ref=kvsend.at[d - 1, b],
                    dst_ref=kvall.at[d - 1, b],
                    send_sem=kv_send_sems.at[d - 1, b],
                    recv_sem=kv_recv_sems.at[d - 1, b],
                    device_id=(dst,),
                    device_id_type=pl.DeviceIdType.MESH,
                )
                r.start()
                kv_rdmas[(d, b)] = r

        wo_bf = wo_ref[:, :].astype(BF)
        wq_bf = wq_ref[:, :].astype(BF)

        def project_q(b):
            qp = lax.dot_general(
                x_ref[b].astype(BF), wq_bf, (((1,), (0,)), ((), ())),
                preferred_element_type=F32).astype(BF)
            pos = 0
            for r in range(3):
                for qb in QCLS_BLOCKS[r]:
                    qbuf[b, pos * BLK:(pos + 1) * BLK, :] = (
                        qp[qb * BLK:(qb + 1) * BLK, :])
                    pos += 1

        def assemble(d, b):
            src = lax.rem(my - d + N_DEV, N_DEV)
            for j in range(NQB):
                g = src * NQB + j
                off = (lax.rem(g, 3) * 11 + g // 3) * BLK
                rows = slice(j * BLK, (j + 1) * BLK)
                if d == 0:
                    kg[b, pl.ds(off, BLK), :] = (
                        k_ref[b, rows, pl.ds(my * HID, HID)].astype(BF))
                    vg[b, pl.ds(off, BLK), :] = (
                        v_ref[b, rows, pl.ds(my * HID, HID)].astype(BF))
                else:
                    kg[b, pl.ds(off, BLK), :] = kvall[d - 1, b, 0, rows, :]
                    vg[b, pl.ds(off, BLK), :] = kvall[d - 1, b, 1, rows, :]

        def attention(b):
            def pair_step(hp, carry, b=b):
                hs = hp * PAIR
                q2 = qbuf[b, :, pl.ds(hs, PAIR)]
                k0 = kg[b, 0:BLK, pl.ds(hs, PAIR)]
                v0 = vg[b, 0:BLK, pl.ds(hs, PAIR)]
                outs = []
                lparts = [[], []]
                cparts = [[], []]
                for r in range(3):
                    r2 = PAIR_RES[r]
                    kr = kg[b, pl.ds(KRES_BASE[r2], KLEN[r2]),
                            pl.ds(hs, PAIR)]
                    vr = vg[b, pl.ds(KRES_BASE[r2], KLEN[r2]),
                            pl.ds(hs, PAIR)]
                    extras = []
                    if r != 0:
                        for qb in QCLS_BLOCKS[r]:
                            drow = (11 * (qb % 3) + qb // 3) * BLK
                            kd = kg[b, pl.ds(drow, BLK), pl.ds(hs, PAIR)]
                            vd = vg[b, pl.ds(drow, BLK), pl.ds(hs, PAIR)]
                            extras.append((kd, vd))
                    for sub in range(2):
                        lo, hi = sub * DH, (sub + 1) * DH
                        qr = q2[QCLS_BASE[r]:QCLS_BASE[r] + QLEN[r], lo:hi]
                        s = lax.dot_general(qr, kr[:, lo:hi],
                                            (((1,), (1,)), ((), ())),
                                            preferred_element_type=F32)
                        p = jnp.exp(s * 0.125)
                        lr = p.sum(axis=1, keepdims=True)
                        cr = lax.dot_general(p.astype(BF), vr[:, lo:hi],
                                             (((1,), (0,)), ((), ())),
                                             preferred_element_type=F32)
                        if r != 0:
                            les, ces = [], []
                            for idx, qb in enumerate(QCLS_BLOCKS[r]):
                                kd, vd = extras[idx]
                                qe = q2[QCLS_BASE[r] + idx * BLK:
                                        QCLS_BASE[r] + (idx + 1) * BLK,
                                        lo:hi]
                                ke = jnp.concatenate(
                                    [k0[:, lo:hi], kd[:, lo:hi]], axis=0)
                                ve = jnp.concatenate(
                                    [v0[:, lo:hi], vd[:, lo:hi]], axis=0)
                                se = lax.dot_general(
                                    qe, ke, (((1,), (1,)), ((), ())),
                                    preferred_element_type=F32)
                                pe = jnp.exp(se * 0.125)
                                les.append(pe.sum(axis=1, keepdims=True))
                                ces.append(lax.dot_general(
                                    pe.astype(BF), ve,
                                    (((1,), (0,)), ((), ())),
                                    preferred_element_type=F32))
                            lr = lr + jnp.concatenate(les, axis=0)
                            cr = cr + jnp.concatenate(ces, axis=0)
                        lparts[sub].append(lr)
                        cparts[sub].append(cr)
                for sub in range(2):
                    ctx_all = jnp.concatenate(cparts[sub], axis=0)
                    l_all = jnp.concatenate(lparts[sub], axis=0)
                    outs.append(ctx_all / l_all)
                cbuf[b, :, pl.ds(hs, PAIR)] = jnp.concatenate(outs, axis=1)
                return carry

            lax.fori_loop(0, H_SH // 2, pair_step, 0)

        o_rdmas = []

        def out_stage(b):
            outp = lax.dot_general(
                cbuf[b].astype(BF), wo_bf, (((1,), (0,)), ((), ())),
                preferred_element_type=F32)
            for qb in range(NQB):
                pp = PERMPOS[qb]
                rows_val = outp[pp * BLK:(pp + 1) * BLK, :]
                out_ref[b, qb * BLK:(qb + 1) * BLK, :] = rows_val
                osend[b, qb * BLK:(qb + 1) * BLK, :] = rows_val.astype(BF)
            for d in range(1, N_DEV):
                dst = lax.rem(my + d, N_DEV)
                r = pltpu.make_async_remote_copy(
                    src_ref=osend.at[b],
                    dst_ref=orecv.at[d - 1, b],
                    send_sem=o_send_sems.at[d - 1, b],
                    recv_sem=o_recv_sems.at[d - 1, b],
                    device_id=(dst,),
                    device_id_type=pl.DeviceIdType.MESH,
                )
                r.start()
                o_rdmas.append(r)

        project_q(0)
        assemble(0, 0)
        project_q(1)
        for d in (1, 3, 2):
            kv_rdmas[(d, 0)].wait_recv()
            assemble(d, 0)
        attention(0)
        out_stage(0)
        assemble(0, 1)
        for d in (1, 3, 2):
            kv_rdmas[(d, 1)].wait_recv()
            assemble(d, 1)
        attention(1)
        out_stage(1)

        for r in kv_rdmas.values():
            r.wait_send()
        for r in o_rdmas:
            r.wait_send()
            r.wait_recv()
        out_ref[:, :, :] = (out_ref[:, :, :]
                            + orecv[0].astype(F32)
                            + orecv[1].astype(F32)
                            + orecv[2].astype(F32))

    return pl.pallas_call(
        body,
        out_shape=jax.ShapeDtypeStruct((B, SQ, D_OUT), F32),
        in_specs=[pl.BlockSpec(memory_space=pltpu.VMEM)] * 5,
        out_specs=pl.BlockSpec(memory_space=pltpu.VMEM),
        scratch_shapes=[
            pltpu.VMEM((N_DEV - 1, B, 2, SKV_SH, HID), BF),
            pltpu.VMEM((N_DEV - 1, B, 2, SKV_SH, HID), BF),
            pltpu.VMEM((B, SKV, HID), BF),
            pltpu.VMEM((B, SKV, HID), BF),
            pltpu.VMEM((B, SQ, HID), BF),
            pltpu.VMEM((B, SQ, HID), F32),
            pltpu.VMEM((B, SQ, D_OUT), BF),
            pltpu.VMEM((N_DEV - 1, B, SQ, D_OUT), BF),
            pltpu.SemaphoreType.DMA((N_DEV - 1, B)),
            pltpu.SemaphoreType.DMA((N_DEV - 1, B)),
            pltpu.SemaphoreType.DMA((N_DEV - 1, B)),
            pltpu.SemaphoreType.DMA((N_DEV - 1, B)),
        ],
        compiler_params=pltpu.CompilerParams(
            vmem_limit_bytes=100 * 1024 * 1024,
        ),
    )(x, Wq, K2, V2, Wo)
